# baseline (speedup 1.0000x reference)
import functools
import math

import jax
import jax.numpy as jnp
from jax import lax
from jax.experimental import pallas as pl
from jax.experimental.pallas import tpu as pltpu
from jax.experimental.pallas import tpu_sc as plsc

N = 10000
D = 128
E = 320000
C = 40
ALPHA = 0.9
LAMBDA = 1.0

NC = 2
NS = 16
NW = NC * NS
NPAD = 10112
RPT = NPAD // NS
DUMMY = N

EPT = E // NW
KD = 288
FLEND = 10368
NCHD = FLEND // KD

H = 5120
LDUMMY = H
MSH = 5248
MRPT = MSH // NS
DRPT = H // NS
K = 128
TCH = 176
TSLOT = TCH * K
PH = 84
WCH = 89
WLEN = WCH * K
NBUF = 4


@functools.cache
def _sc_mesh():
    return plsc.VectorSubcoreMesh(core_axis_name="c", subcore_axis_name="s")


@functools.cache
def _sc_deg_fn():
    @functools.partial(
        pl.kernel,
        out_type=jax.ShapeDtypeStruct((NC, NPAD, D), jnp.float32),
        mesh=_sc_mesh(),
        scratch_types=[
            pltpu.VMEM_SHARED((NPAD, D), jnp.float32),
            pltpu.VMEM((FLEND,), jnp.int32),
            pltpu.VMEM((KD, D), jnp.float32),
            pltpu.SemaphoreType.DMA,
        ],
    )
    def deg_kernel(dstf, ones_h, zeros_h, deg_out, deg_sh, dst_v, ones_v,
                   sem):
        c = lax.axis_index("c")
        s = lax.axis_index("s")
        wid = s * NC + c
        pltpu.sync_copy(zeros_h.at[pl.ds(s * RPT, RPT)],
                        deg_sh.at[pl.ds(s * RPT, RPT)])
        pltpu.sync_copy(dstf.at[pl.ds(wid * FLEND, FLEND)], dst_v)
        pltpu.sync_copy(ones_h, ones_v)
        plsc.subcore_barrier()

        @pl.loop(0, NCHD)
        def _(j):
            pltpu.async_copy(ones_v, deg_sh.at[dst_v.at[pl.ds(j * KD, KD)]],
                             sem, add=True)

        @pl.loop(0, NCHD)
        def _(j):
            pltpu.make_async_copy(ones_v,
                                  deg_sh.at[dst_v.at[pl.ds(0, KD)]],
                                  sem).wait()

        plsc.subcore_barrier()
        pltpu.sync_copy(deg_sh.at[pl.ds(s * RPT, RPT)],
                        deg_out.at[c, pl.ds(s * RPT, RPT)])

    return deg_kernel


@functools.cache
def _sc_layer_fn():
    @functools.partial(
        pl.kernel,
        out_type=jax.ShapeDtypeStruct((NC * H, D), jnp.float32),
        mesh=_sc_mesh(),
        scratch_types=(
            [pltpu.VMEM_SHARED((MSH, D), jnp.float32),
             pltpu.VMEM((WLEN,), jnp.int32),
             pltpu.VMEM((WLEN,), jnp.int32)]
            + [pltpu.VMEM((K, D), jnp.float32)] * NBUF
            + [pltpu.SemaphoreType.DMA] * NBUF
            + [pltpu.VMEM((128,), jnp.int32)]
            + [pltpu.SMEM((8,), jnp.int32)]
        ),
    )
    def layer_kernel(hs, srcp, dstp, counts, zeros_h, m_out,
                     m_sh, src_v, dst_v, *rest):
        bufs = rest[:NBUF]
        gsems = rest[NBUF:2 * NBUF]
        cnt_v = rest[2 * NBUF]
        cnt_s = rest[2 * NBUF + 1]
        c = lax.axis_index("c")
        s = lax.axis_index("s")
        wid = c * NS + s
        pltpu.sync_copy(zeros_h.at[pl.ds(s * MRPT, MRPT)],
                        m_sh.at[pl.ds(s * MRPT, MRPT)])
        del cnt_s
        pltpu.sync_copy(counts.at[c], cnt_v)
        plsc.subcore_barrier()
        nch = cnt_v[pl.ds(0, 16)][0]

        def gather(lj, b):
            pltpu.async_copy(hs.at[src_v.at[pl.ds(lj * K, K)]],
                             bufs[b], gsems[b])

        def wait_gather(b):
            pltpu.make_async_copy(hs.at[src_v.at[pl.ds(0, K)]],
                                  bufs[b], gsems[b]).wait()

        for p in range(2):
            pltpu.sync_copy(
                srcp.at[pl.ds(wid * TSLOT + p * (PH * K), WLEN)], src_v)
            pltpu.sync_copy(
                dstp.at[pl.ds(wid * TSLOT + p * (PH * K), WLEN)], dst_v)
            cnt = lax.max(lax.min(nch - p * PH, PH), 0)
            ng = (cnt + 3) // 4
            for lj in range(3):
                gather(lj, lj)

            @pl.loop(0, ng)
            def _(jj):
                for b0 in range(4):
                    lj = 4 * jj + b0
                    wait_gather(b0)
                    pltpu.sync_copy(bufs[b0],
                                    m_sh.at[dst_v.at[pl.ds(lj * K, K)]],
                                    add=True)
                    gather(lj + 3, (b0 + 3) % 4)

            for b in range(3):
                wait_gather(b)

        plsc.subcore_barrier()
        pltpu.sync_copy(m_sh.at[pl.ds(s * DRPT, DRPT)],
                        m_out.at[pl.ds(c * H + s * DRPT, DRPT)])

    return layer_kernel


BLK = 1000


def _tc_prep(deg2, x):
    def body(deg_ref, x_ref, norm_ref, hs_ref):
        d = deg_ref[0, :, 0:1] + deg_ref[1, :, 0:1]
        nrm = lax.rsqrt(jnp.maximum(d, 1.0))
        nb = jnp.broadcast_to(nrm, (BLK, D))
        norm_ref[...] = nb
        hs_ref[...] = x_ref[...] * nb

    return pl.pallas_call(
        body,
        grid=(N // BLK,),
        in_specs=[
            pl.BlockSpec((NC, BLK, D), lambda j: (0, j, 0)),
            pl.BlockSpec((BLK, D), lambda j: (j, 0)),
        ],
        out_specs=[
            pl.BlockSpec((BLK, D), lambda j: (j, 0)),
            pl.BlockSpec((BLK, D), lambda j: (j, 0)),
        ],
        out_shape=[
            jax.ShapeDtypeStruct((N, D), jnp.float32),
            jax.ShapeDtypeStruct((NPAD, D), jnp.float32),
        ],
    )(deg2, x)


def _tc_layer(m2, x, normb, W, beta):
    def body(m_ref, x_ref, n_ref, w_ref, hs_ref):
        mcat = m_ref[...]
        nb = n_ref[...]
        g = mcat * nb * (1.0 - ALPHA) + ALPHA * x_ref[...]
        hw = jnp.dot(g, w_ref[...], preferred_element_type=jnp.float32)
        h = jnp.maximum((1.0 - beta) * g + beta * hw, 0.0)
        hs_ref[...] = h * nb

    return pl.pallas_call(
        body,
        grid=(N // BLK,),
        in_specs=[
            pl.BlockSpec((BLK, D), lambda j: (j, 0)),
            pl.BlockSpec((BLK, D), lambda j: (j, 0)),
            pl.BlockSpec((BLK, D), lambda j: (j, 0)),
            pl.BlockSpec((D, D), lambda j: (0, 0)),
        ],
        out_specs=pl.BlockSpec((BLK, D), lambda j: (j, 0)),
        out_shape=jax.ShapeDtypeStruct((NPAD, D), jnp.float32),
    )(m2, x, normb, W)


def _tc_final(m2, x, normb, W, Wfc, bfc2, beta):
    def body(m_ref, x_ref, n_ref, w_ref, wfc_ref, b_ref, out_ref):
        mcat = m_ref[...]
        nb = n_ref[...]
        g = mcat * nb * (1.0 - ALPHA) + ALPHA * x_ref[...]
        hw = jnp.dot(g, w_ref[...], preferred_element_type=jnp.float32)
        h = jnp.maximum((1.0 - beta) * g + beta * hw, 0.0)
        out_ref[...] = (jnp.dot(h, wfc_ref[...],
                                preferred_element_type=jnp.float32)
                        + b_ref[...])

    return pl.pallas_call(
        body,
        grid=(N // BLK,),
        in_specs=[
            pl.BlockSpec((BLK, D), lambda j: (j, 0)),
            pl.BlockSpec((BLK, D), lambda j: (j, 0)),
            pl.BlockSpec((BLK, D), lambda j: (j, 0)),
            pl.BlockSpec((D, D), lambda j: (0, 0)),
            pl.BlockSpec((D, C), lambda j: (0, 0)),
            pl.BlockSpec((1, C), lambda j: (0, 0)),
        ],
        out_specs=pl.BlockSpec((BLK, C), lambda j: (j, 0)),
        out_shape=jax.ShapeDtypeStruct((N, C), jnp.float32),
    )(m2, x, normb, W, Wfc, bfc2)


def kernel(x, edge_index, W1, W2, W3, W4, W5, Wfc, bfc):
    src = edge_index[0].astype(jnp.int32)
    dst = edge_index[1].astype(jnp.int32)
    dstf = jnp.pad(dst.reshape(NW, EPT), ((0, 0), (0, FLEND - EPT)),
                   constant_values=DUMMY).reshape(NW * FLEND)

    key = (dst >= H).astype(jnp.int32)
    order = jnp.argsort(key, stable=True)
    ssrc = jnp.take(src, order)
    sdst = jnp.take(dst, order)
    nlo = E - jnp.sum(key)
    i = jnp.arange(TSLOT, dtype=jnp.int32)[None, :]
    t = jnp.arange(NS, dtype=jnp.int32)[:, None]

    def build(base, n, node_base):
        stripe = (n + NS - 1) // NS
        g = base + t * stripe + i
        v = (i < stripe) & (g < base + n)
        gc = jnp.clip(g, 0, E - 1)
        sa = jnp.where(v, jnp.take(ssrc, gc), DUMMY)
        da = jnp.where(v, jnp.take(sdst, gc) - node_base, LDUMMY)
        return sa, da, (stripe + K - 1) // K

    s0, d0, nch0 = build(0, nlo, 0)
    s1, d1, nch1 = build(nlo, E - nlo, H)
    srcp = jnp.stack([s0, s1]).reshape(NC * NS * TSLOT)
    dstp = jnp.stack([d0, d1]).reshape(NC * NS * TSLOT)
    counts = jnp.stack([jnp.full((128,), nch0, jnp.int32),
                    jnp.full((128,), nch1, jnp.int32)])

    zerosd = jnp.zeros((NPAD, D), jnp.float32)
    onesd = jnp.ones((KD, D), jnp.float32)

    deg2 = _sc_deg_fn()(dstf, onesd, zerosd)
    normb, hs = _tc_prep(deg2, x)

    Ws = (W1, W2, W3, W4, W5)
    for i2 in range(4):
        beta = math.log(LAMBDA / (i2 + 1) + 1.0)
        m2 = _sc_layer_fn()(hs, srcp, dstp, counts, zerosd)
        hs = _tc_layer(m2, x, normb, Ws[i2], beta)
    beta = math.log(LAMBDA / 5.0 + 1.0)
    m2 = _sc_layer_fn()(hs, srcp, dstp, counts, zerosd)
    return _tc_final(m2, x, normb, Ws[4], Wfc, bfc.reshape(1, C), beta)


# speedup vs baseline: 3.6173x; 3.6173x over previous
import functools
import math

import jax
import jax.numpy as jnp
from jax import lax
from jax.experimental import pallas as pl
from jax.experimental.pallas import tpu as pltpu
from jax.experimental.pallas import tpu_sc as plsc

N = 10000
D = 128
E = 320000
C = 40
ALPHA = 0.9
LAMBDA = 1.0

NC = 2
NS = 16
NW = NC * NS
NPAD = 10112
RPT = NPAD // NS
DUMMY = N

CH = 128
EPT = E // NW
CAP = 80
KD = 288
FLEND = 10368
NCHD = FLEND // KD


@functools.cache
def _sc_mesh():
    return plsc.VectorSubcoreMesh(core_axis_name="c", subcore_axis_name="s")


@functools.cache
def _sc_deg_fn():
    @functools.partial(
        pl.kernel,
        out_type=jax.ShapeDtypeStruct((NC, NPAD, D), jnp.float32),
        mesh=_sc_mesh(),
        scratch_types=[
            pltpu.VMEM_SHARED((NPAD, D), jnp.float32),
            pltpu.VMEM((FLEND,), jnp.int32),
            pltpu.VMEM((KD, D), jnp.float32),
            pltpu.SemaphoreType.DMA,
        ],
    )
    def deg_kernel(dstf, ones_h, zeros_h, deg_out, deg_sh, dst_v, ones_v,
                   sem):
        c = lax.axis_index("c")
        s = lax.axis_index("s")
        wid = s * NC + c
        pltpu.sync_copy(zeros_h.at[pl.ds(s * RPT, RPT)],
                        deg_sh.at[pl.ds(s * RPT, RPT)])
        pltpu.sync_copy(dstf.at[pl.ds(wid * FLEND, FLEND)], dst_v)
        pltpu.sync_copy(ones_h, ones_v)
        plsc.subcore_barrier()

        @pl.loop(0, NCHD)
        def _(j):
            pltpu.async_copy(ones_v, deg_sh.at[dst_v.at[pl.ds(j * KD, KD)]],
                             sem, add=True)

        @pl.loop(0, NCHD)
        def _(j):
            pltpu.make_async_copy(ones_v,
                                  deg_sh.at[dst_v.at[pl.ds(0, KD)]],
                                  sem).wait()

        plsc.subcore_barrier()
        pltpu.sync_copy(deg_sh.at[pl.ds(s * RPT, RPT)],
                        deg_out.at[c, pl.ds(s * RPT, RPT)])

    return deg_kernel


@functools.cache
def _sc_layer_fn():
    @functools.partial(
        pl.kernel,
        out_type=jax.ShapeDtypeStruct((NC, NPAD, D), jnp.float32),
        mesh=_sc_mesh(),
        scratch_types=[
            pltpu.VMEM_SHARED((NPAD, D), jnp.float32),
            pltpu.VMEM((CAP, CH), jnp.int32),
            pltpu.VMEM((CAP, CH), jnp.int32),
            pltpu.VMEM((CH, D), jnp.float32),
            pltpu.SemaphoreType.DMA,
        ],
    )
    def layer_kernel(hs, srcp, dstp, zeros_h, m_out,
                     m_sh, src_v, dst_v, buf, sem):
        c = lax.axis_index("c")
        s = lax.axis_index("s")
        wid = s * NC + c
        pltpu.sync_copy(zeros_h.at[pl.ds(s * RPT, RPT)],
                        m_sh.at[pl.ds(s * RPT, RPT)])
        pltpu.sync_copy(srcp.at[wid], src_v)
        pltpu.sync_copy(dstp.at[wid], dst_v)
        plsc.subcore_barrier()

        @pl.loop(0, CAP)
        def _(j):
            pltpu.async_copy(hs.at[src_v.at[j]], buf, sem).wait()
            pltpu.sync_copy(buf, m_sh.at[dst_v.at[j]], add=True)

        plsc.subcore_barrier()
        pltpu.sync_copy(m_sh.at[pl.ds(s * RPT, RPT)],
                        m_out.at[c, pl.ds(s * RPT, RPT)])

    return layer_kernel


BLK = 1000


def _tc_prep(deg2, x):
    def body(deg_ref, x_ref, norm_ref, hs_ref):
        d = deg_ref[0, :, 0:1] + deg_ref[1, :, 0:1]
        nrm = lax.rsqrt(jnp.maximum(d, 1.0))
        nb = jnp.broadcast_to(nrm, (BLK, D))
        norm_ref[...] = nb
        hs_ref[...] = x_ref[...] * nb

    return pl.pallas_call(
        body,
        grid=(N // BLK,),
        in_specs=[
            pl.BlockSpec((NC, BLK, D), lambda j: (0, j, 0)),
            pl.BlockSpec((BLK, D), lambda j: (j, 0)),
        ],
        out_specs=[
            pl.BlockSpec((BLK, D), lambda j: (j, 0)),
            pl.BlockSpec((BLK, D), lambda j: (j, 0)),
        ],
        out_shape=[
            jax.ShapeDtypeStruct((N, D), jnp.float32),
            jax.ShapeDtypeStruct((NPAD, D), jnp.float32),
        ],
    )(deg2, x)


def _tc_layer(m2, x, normb, W, beta):
    def body(m_ref, x_ref, n_ref, w_ref, hs_ref):
        mcat = m_ref[0] + m_ref[1]
        nb = n_ref[...]
        g = mcat * nb * (1.0 - ALPHA) + ALPHA * x_ref[...]
        hw = jnp.dot(g, w_ref[...], preferred_element_type=jnp.float32)
        h = jnp.maximum((1.0 - beta) * g + beta * hw, 0.0)
        hs_ref[...] = h * nb

    return pl.pallas_call(
        body,
        grid=(N // BLK,),
        in_specs=[
            pl.BlockSpec((NC, BLK, D), lambda j: (0, j, 0)),
            pl.BlockSpec((BLK, D), lambda j: (j, 0)),
            pl.BlockSpec((BLK, D), lambda j: (j, 0)),
            pl.BlockSpec((D, D), lambda j: (0, 0)),
        ],
        out_specs=pl.BlockSpec((BLK, D), lambda j: (j, 0)),
        out_shape=jax.ShapeDtypeStruct((NPAD, D), jnp.float32),
    )(m2, x, normb, W)


def _tc_final(m2, x, normb, W, Wfc, bfc2, beta):
    def body(m_ref, x_ref, n_ref, w_ref, wfc_ref, b_ref, out_ref):
        mcat = m_ref[0] + m_ref[1]
        nb = n_ref[...]
        g = mcat * nb * (1.0 - ALPHA) + ALPHA * x_ref[...]
        hw = jnp.dot(g, w_ref[...], preferred_element_type=jnp.float32)
        h = jnp.maximum((1.0 - beta) * g + beta * hw, 0.0)
        out_ref[...] = (jnp.dot(h, wfc_ref[...],
                                preferred_element_type=jnp.float32)
                        + b_ref[...])

    return pl.pallas_call(
        body,
        grid=(N // BLK,),
        in_specs=[
            pl.BlockSpec((NC, BLK, D), lambda j: (0, j, 0)),
            pl.BlockSpec((BLK, D), lambda j: (j, 0)),
            pl.BlockSpec((BLK, D), lambda j: (j, 0)),
            pl.BlockSpec((D, D), lambda j: (0, 0)),
            pl.BlockSpec((D, C), lambda j: (0, 0)),
            pl.BlockSpec((1, C), lambda j: (0, 0)),
        ],
        out_specs=pl.BlockSpec((BLK, C), lambda j: (j, 0)),
        out_shape=jax.ShapeDtypeStruct((N, C), jnp.float32),
    )(m2, x, normb, W, Wfc, bfc2)


def kernel(x, edge_index, W1, W2, W3, W4, W5, Wfc, bfc):
    src = edge_index[0].astype(jnp.int32)
    dst = edge_index[1].astype(jnp.int32)
    srcp = jnp.pad(src.reshape(NW, EPT), ((0, 0), (0, CAP * CH - EPT)),
                   constant_values=DUMMY).reshape(NW, CAP, CH)
    dstp = jnp.pad(dst.reshape(NW, EPT), ((0, 0), (0, CAP * CH - EPT)),
                   constant_values=DUMMY).reshape(NW, CAP, CH)
    dstf = jnp.pad(dst.reshape(NW, EPT), ((0, 0), (0, FLEND - EPT)),
                   constant_values=DUMMY).reshape(NW * FLEND)
    zerosd = jnp.zeros((NPAD, D), jnp.float32)
    onesd = jnp.ones((KD, D), jnp.float32)

    deg2 = _sc_deg_fn()(dstf, onesd, zerosd)
    normb, hs = _tc_prep(deg2, x)

    Ws = (W1, W2, W3, W4, W5)
    for i in range(4):
        beta = math.log(LAMBDA / (i + 1) + 1.0)
        m2 = _sc_layer_fn()(hs, srcp, dstp, zerosd)
        hs = _tc_layer(m2, x, normb, Ws[i], beta)
    beta = math.log(LAMBDA / 5.0 + 1.0)
    m2 = _sc_layer_fn()(hs, srcp, dstp, zerosd)
    return _tc_final(m2, x, normb, Ws[4], Wfc, bfc.reshape(1, C), beta)


# speedup vs baseline: 5.2705x; 1.4570x over previous
import functools
import math

import jax
import jax.numpy as jnp
from jax import lax
from jax.experimental import pallas as pl
from jax.experimental.pallas import tpu as pltpu
from jax.experimental.pallas import tpu_sc as plsc

N = 10000
D = 128
E = 320000
C = 40
ALPHA = 0.9
LAMBDA = 1.0

NC = 2
NS = 16
NW = NC * NS
NPAD = 10112
RPT = NPAD // NS
DUMMY = N

CH = 128
EPT = E // NW
CAP = 79
KD = 288
FLEND = 10368
NCHD = FLEND // KD


@functools.cache
def _sc_mesh():
    return plsc.VectorSubcoreMesh(core_axis_name="c", subcore_axis_name="s")


@functools.cache
def _sc_deg_fn():
    @functools.partial(
        pl.kernel,
        out_type=jax.ShapeDtypeStruct((NC, NPAD, D), jnp.float32),
        mesh=_sc_mesh(),
        scratch_types=[
            pltpu.VMEM_SHARED((NPAD, D), jnp.float32),
            pltpu.VMEM((FLEND,), jnp.int32),
            pltpu.VMEM((KD, D), jnp.float32),
            pltpu.SemaphoreType.DMA,
        ],
    )
    def deg_kernel(dstf, ones_h, zeros_h, deg_out, deg_sh, dst_v, ones_v,
                   sem):
        c = lax.axis_index("c")
        s = lax.axis_index("s")
        wid = s * NC + c
        pltpu.sync_copy(zeros_h.at[pl.ds(s * RPT, RPT)],
                        deg_sh.at[pl.ds(s * RPT, RPT)])
        pltpu.sync_copy(dstf.at[pl.ds(wid * FLEND, FLEND)], dst_v)
        pltpu.sync_copy(ones_h, ones_v)
        plsc.subcore_barrier()

        @pl.loop(0, NCHD)
        def _(j):
            pltpu.async_copy(ones_v, deg_sh.at[dst_v.at[pl.ds(j * KD, KD)]],
                             sem, add=True)

        @pl.loop(0, NCHD)
        def _(j):
            pltpu.make_async_copy(ones_v,
                                  deg_sh.at[dst_v.at[pl.ds(0, KD)]],
                                  sem).wait()

        plsc.subcore_barrier()
        pltpu.sync_copy(deg_sh.at[pl.ds(s * RPT, RPT)],
                        deg_out.at[c, pl.ds(s * RPT, RPT)])

    return deg_kernel


@functools.cache
def _sc_layer_fn():
    @functools.partial(
        pl.kernel,
        out_type=jax.ShapeDtypeStruct((NC, NPAD, D), jnp.float32),
        mesh=_sc_mesh(),
        scratch_types=[
            pltpu.VMEM_SHARED((NPAD, D), jnp.float32),
            pltpu.VMEM((CAP, CH), jnp.int32),
            pltpu.VMEM((CAP, CH), jnp.int32),
            pltpu.VMEM((CH, D), jnp.float32),
            pltpu.SemaphoreType.DMA,
        ],
    )
    def layer_kernel(hs, srcp, dstp, zeros_h, m_out,
                     m_sh, src_v, dst_v, buf, sem):
        c = lax.axis_index("c")
        s = lax.axis_index("s")
        wid = s * NC + c
        pltpu.sync_copy(zeros_h.at[pl.ds(s * RPT, RPT)],
                        m_sh.at[pl.ds(s * RPT, RPT)])
        pltpu.sync_copy(srcp.at[wid], src_v)
        pltpu.sync_copy(dstp.at[wid], dst_v)
        plsc.subcore_barrier()

        @pl.loop(0, CAP)
        def _(j):
            pltpu.async_copy(hs.at[src_v.at[j]], buf, sem).wait()
            pltpu.sync_copy(buf, m_sh.at[dst_v.at[j]], add=True)

        plsc.subcore_barrier()
        pltpu.sync_copy(m_sh.at[pl.ds(s * RPT, RPT)],
                        m_out.at[c, pl.ds(s * RPT, RPT)])

    return layer_kernel


BLK = 1000


def _tc_prep(deg2, x):
    def body(deg_ref, x_ref, norm_ref, hs_ref):
        d = deg_ref[0, :, 0:1] + deg_ref[1, :, 0:1]
        nrm = lax.rsqrt(jnp.maximum(d, 1.0))
        nb = jnp.broadcast_to(nrm, (BLK, D))
        norm_ref[...] = nb
        hs_ref[...] = x_ref[...] * nb

    return pl.pallas_call(
        body,
        grid=(N // BLK,),
        in_specs=[
            pl.BlockSpec((NC, BLK, D), lambda j: (0, j, 0)),
            pl.BlockSpec((BLK, D), lambda j: (j, 0)),
        ],
        out_specs=[
            pl.BlockSpec((BLK, D), lambda j: (j, 0)),
            pl.BlockSpec((BLK, D), lambda j: (j, 0)),
        ],
        out_shape=[
            jax.ShapeDtypeStruct((N, D), jnp.float32),
            jax.ShapeDtypeStruct((NPAD, D), jnp.float32),
        ],
    )(deg2, x)


def _tc_layer(m2, x, normb, W, beta):
    def body(m_ref, x_ref, n_ref, w_ref, hs_ref):
        mcat = m_ref[0] + m_ref[1]
        nb = n_ref[...]
        g = mcat * nb * (1.0 - ALPHA) + ALPHA * x_ref[...]
        hw = jnp.dot(g, w_ref[...], preferred_element_type=jnp.float32)
        h = jnp.maximum((1.0 - beta) * g + beta * hw, 0.0)
        hs_ref[...] = h * nb

    return pl.pallas_call(
        body,
        grid=(N // BLK,),
        in_specs=[
            pl.BlockSpec((NC, BLK, D), lambda j: (0, j, 0)),
            pl.BlockSpec((BLK, D), lambda j: (j, 0)),
            pl.BlockSpec((BLK, D), lambda j: (j, 0)),
            pl.BlockSpec((D, D), lambda j: (0, 0)),
        ],
        out_specs=pl.BlockSpec((BLK, D), lambda j: (j, 0)),
        out_shape=jax.ShapeDtypeStruct((NPAD, D), jnp.float32),
    )(m2, x, normb, W)


def _tc_final(m2, x, normb, W, Wfc, bfc2, beta):
    def body(m_ref, x_ref, n_ref, w_ref, wfc_ref, b_ref, out_ref):
        mcat = m_ref[0] + m_ref[1]
        nb = n_ref[...]
        g = mcat * nb * (1.0 - ALPHA) + ALPHA * x_ref[...]
        hw = jnp.dot(g, w_ref[...], preferred_element_type=jnp.float32)
        h = jnp.maximum((1.0 - beta) * g + beta * hw, 0.0)
        out_ref[...] = (jnp.dot(h, wfc_ref[...],
                                preferred_element_type=jnp.float32)
                        + b_ref[...])

    return pl.pallas_call(
        body,
        grid=(N // BLK,),
        in_specs=[
            pl.BlockSpec((NC, BLK, D), lambda j: (0, j, 0)),
            pl.BlockSpec((BLK, D), lambda j: (j, 0)),
            pl.BlockSpec((BLK, D), lambda j: (j, 0)),
            pl.BlockSpec((D, D), lambda j: (0, 0)),
            pl.BlockSpec((D, C), lambda j: (0, 0)),
            pl.BlockSpec((1, C), lambda j: (0, 0)),
        ],
        out_specs=pl.BlockSpec((BLK, C), lambda j: (j, 0)),
        out_shape=jax.ShapeDtypeStruct((N, C), jnp.float32),
    )(m2, x, normb, W, Wfc, bfc2)


def kernel(x, edge_index, W1, W2, W3, W4, W5, Wfc, bfc):
    src = edge_index[0].astype(jnp.int32)
    dst = edge_index[1].astype(jnp.int32)
    srcp = jnp.pad(src.reshape(NW, EPT), ((0, 0), (0, CAP * CH - EPT)),
                   constant_values=DUMMY).reshape(NW, CAP, CH)
    dstp = jnp.pad(dst.reshape(NW, EPT), ((0, 0), (0, CAP * CH - EPT)),
                   constant_values=DUMMY).reshape(NW, CAP, CH)
    dstf = jnp.pad(dst.reshape(NW, EPT), ((0, 0), (0, FLEND - EPT)),
                   constant_values=DUMMY).reshape(NW * FLEND)
    zerosd = jnp.zeros((NPAD, D), jnp.float32)
    onesd = jnp.ones((KD, D), jnp.float32)

    deg2 = _sc_deg_fn()(dstf, onesd, zerosd)
    normb, hs = _tc_prep(deg2, x)

    Ws = (W1, W2, W3, W4, W5)
    for i in range(4):
        beta = math.log(LAMBDA / (i + 1) + 1.0)
        m2 = _sc_layer_fn()(hs, srcp, dstp, zerosd)
        hs = _tc_layer(m2, x, normb, Ws[i], beta)
    beta = math.log(LAMBDA / 5.0 + 1.0)
    m2 = _sc_layer_fn()(hs, srcp, dstp, zerosd)
    return _tc_final(m2, x, normb, Ws[4], Wfc, bfc.reshape(1, C), beta)
